# baseline (speedup 1.0000x reference)
import functools

import jax
import jax.numpy as jnp
from jax import lax
from jax.experimental import pallas as pl
from jax.experimental.pallas import tpu as pltpu
from jax.experimental.pallas import tpu_sc as plsc


def _argmin_body(k_codes, bb, x_ref, w_ref, idx_ref, wt_ref):
    w = w_ref[...]
    e2 = jnp.sum(w * w, axis=0)
    for j in range(bb):
        xb = x_ref[j]
        st = lax.dot_general(w, xb, (((0,), (0,)), ((), ())),
                             preferred_element_type=jnp.float32)
        x2 = jnp.sum(xb * xb, axis=0)
        dist = (x2[None, :] - 2.0 * st) + e2[:, None]
        m = jnp.min(dist, axis=0, keepdims=True)
        kiota = lax.broadcasted_iota(jnp.int32, dist.shape, 0)
        am = jnp.min(jnp.where(dist == m, kiota, k_codes), axis=0)
        idx_ref[j, 0, :] = am

    @pl.when(pl.program_id(0) == 0)
    def _():
        wt_ref[...] = w.T


def _argmin_call(x3, weight):
    b, d, p = x3.shape
    k = weight.shape[1]
    bb = 2
    return pl.pallas_call(
        functools.partial(_argmin_body, k, bb),
        grid=(b // bb,),
        in_specs=[
            pl.BlockSpec((bb, d, p), lambda i: (i, 0, 0)),
            pl.BlockSpec((d, k), lambda i: (0, 0)),
        ],
        out_specs=[
            pl.BlockSpec((bb, 1, p), lambda i: (i, 0, 0)),
            pl.BlockSpec((k, d), lambda i: (0, 0)),
        ],
        out_shape=[
            jax.ShapeDtypeStruct((b, 1, p), jnp.int32),
            jax.ShapeDtypeStruct((k, d), jnp.float32),
        ],
    )(x3, weight)


def _sc_gather(wt, idx):
    nc, ns = 2, 16
    nw = nc * ns
    n, d = idx.shape[0], wt.shape[1]
    b_per_w = n // nw
    ch = min(128, b_per_w)
    n_ch = b_per_w // ch
    mesh = plsc.VectorSubcoreMesh(core_axis_name="c", subcore_axis_name="s",
                                  num_cores=nc, num_subcores=ns)

    nbuf = min(3, n_ch)

    @functools.partial(
        pl.kernel, mesh=mesh,
        out_type=jax.ShapeDtypeStruct((n, d), jnp.float32),
        scratch_types=(
            [pltpu.VMEM((ch,), jnp.int32)] * n_ch
            + [pltpu.VMEM((ch, d), jnp.float32)] * nbuf
            + [pltpu.SemaphoreType.DMA] * (2 * nbuf)
        ),
    )
    def gather_kernel(table_hbm, idx_hbm, out_hbm, *bufs_sems):
        idx_bufs = bufs_sems[:n_ch]
        rows_bufs = bufs_sems[n_ch:n_ch + nbuf]
        gsems = bufs_sems[n_ch + nbuf:n_ch + 2 * nbuf]
        osems = bufs_sems[n_ch + 2 * nbuf:]
        wid = lax.axis_index("s") * nc + lax.axis_index("c")
        base = wid * b_per_w
        for c in range(n_ch):
            pltpu.sync_copy(idx_hbm.at[pl.ds(base + c * ch, ch)], idx_bufs[c])
        gathers = [None] * n_ch
        outs = [None] * n_ch
        for c in range(n_ch):
            if c >= nbuf:
                outs[c - nbuf].wait()
            gathers[c] = pltpu.async_copy(
                table_hbm.at[idx_bufs[c]],
                rows_bufs[c % nbuf], gsems[c % nbuf])
            if c >= 1:
                gathers[c - 1].wait()
                outs[c - 1] = pltpu.async_copy(
                    rows_bufs[(c - 1) % nbuf],
                    out_hbm.at[pl.ds(base + (c - 1) * ch, ch)],
                    osems[(c - 1) % nbuf])
        gathers[n_ch - 1].wait()
        outs[n_ch - 1] = pltpu.async_copy(
            rows_bufs[(n_ch - 1) % nbuf],
            out_hbm.at[pl.ds(base + (n_ch - 1) * ch, ch)],
            osems[(n_ch - 1) % nbuf])
        for c in range(max(0, n_ch - nbuf), n_ch):
            outs[c].wait()

    return gather_kernel(wt, idx)


def _transpose_body(q_ref, o_ref):
    o_ref[...] = jnp.transpose(q_ref[...], (0, 2, 1))


def _transpose_call(q3):
    b, p, d = q3.shape
    bs = 8
    return pl.pallas_call(
        _transpose_body,
        grid=(b // bs,),
        in_specs=[pl.BlockSpec((bs, p, d), lambda i: (i, 0, 0))],
        out_specs=pl.BlockSpec((bs, d, p), lambda i: (i, 0, 0)),
        out_shape=jax.ShapeDtypeStruct((b, d, p), jnp.float32),
    )(q3)


def kernel(x, weight):
    b, d, h, w = x.shape
    p = h * w
    x3 = x.reshape(b, d, p)
    idx3, wt = _argmin_call(x3, weight)
    quant = _sc_gather(wt, idx3.reshape(b * p))
    res3 = _transpose_call(x3)
    tie = (quant[0, 0] * 0.0).astype(jnp.int32)
    return (res3.reshape(b, d, h, w),
            idx3.reshape(b, h, w) + tie)


# speedup vs baseline: 1.2891x; 1.2891x over previous
import functools

import jax
import jax.numpy as jnp
from jax import lax
from jax.experimental import pallas as pl
from jax.experimental.pallas import tpu as pltpu
from jax.experimental.pallas import tpu_sc as plsc


def _wt_body(w_ref, wt_ref):
    wt_ref[...] = w_ref[...].T


def _wt_call(weight):
    d, k = weight.shape
    return pl.pallas_call(
        _wt_body,
        out_shape=jax.ShapeDtypeStruct((k, d), jnp.float32),
    )(weight)


def _argmin_body(k_codes, bb, x_ref, wt_ref, idx_ref):
    wt = wt_ref[...]
    e2 = jnp.sum(wt * wt, axis=1)
    for j in range(bb):
        xb = x_ref[j]
        st = lax.dot_general(wt, xb, (((1,), (0,)), ((), ())),
                             preferred_element_type=jnp.float32)
        x2 = jnp.sum(xb * xb, axis=0)
        dist = (x2[None, :] - 2.0 * st) + e2[:, None]
        m = jnp.min(dist, axis=0, keepdims=True)
        kiota = lax.broadcasted_iota(jnp.int32, dist.shape, 0)
        am = jnp.min(jnp.where(dist == m, kiota, k_codes), axis=0)
        idx_ref[j, 0, :] = am


def _argmin_call(x3, wt):
    b, d, p = x3.shape
    k = wt.shape[0]
    bb = 2
    return pl.pallas_call(
        functools.partial(_argmin_body, k, bb),
        grid=(b // bb,),
        in_specs=[
            pl.BlockSpec((bb, d, p), lambda i: (i, 0, 0)),
            pl.BlockSpec((k, d), lambda i: (0, 0)),
        ],
        out_specs=pl.BlockSpec((bb, 1, p), lambda i: (i, 0, 0)),
        out_shape=jax.ShapeDtypeStruct((b, 1, p), jnp.int32),
    )(x3, wt)


def _sc_gather(wt, idx):
    nc, ns = 2, 16
    nw = nc * ns
    n, d = idx.shape[0], wt.shape[1]
    b_per_w = n // nw
    ch = min(128, b_per_w)
    n_ch = b_per_w // ch
    mesh = plsc.VectorSubcoreMesh(core_axis_name="c", subcore_axis_name="s",
                                  num_cores=nc, num_subcores=ns)

    nbuf = min(3, n_ch)

    @functools.partial(
        pl.kernel, mesh=mesh,
        out_type=jax.ShapeDtypeStruct((n, d), jnp.float32),
        scratch_types=(
            [pltpu.VMEM((ch,), jnp.int32)] * n_ch
            + [pltpu.VMEM((ch, d), jnp.float32)] * nbuf
            + [pltpu.SemaphoreType.DMA] * (2 * nbuf)
        ),
    )
    def gather_kernel(table_hbm, idx_hbm, out_hbm, *bufs_sems):
        idx_bufs = bufs_sems[:n_ch]
        rows_bufs = bufs_sems[n_ch:n_ch + nbuf]
        gsems = bufs_sems[n_ch + nbuf:n_ch + 2 * nbuf]
        osems = bufs_sems[n_ch + 2 * nbuf:]
        wid = lax.axis_index("s") * nc + lax.axis_index("c")
        base = wid * b_per_w
        for c in range(n_ch):
            pltpu.sync_copy(idx_hbm.at[pl.ds(base + c * ch, ch)], idx_bufs[c])
        gathers = [None] * n_ch
        outs = [None] * n_ch
        for c in range(n_ch):
            if c >= nbuf:
                outs[c - nbuf].wait()
            gathers[c] = pltpu.async_copy(
                table_hbm.at[idx_bufs[c]],
                rows_bufs[c % nbuf], gsems[c % nbuf])
            if c >= 1:
                gathers[c - 1].wait()
                outs[c - 1] = pltpu.async_copy(
                    rows_bufs[(c - 1) % nbuf],
                    out_hbm.at[pl.ds(base + (c - 1) * ch, ch)],
                    osems[(c - 1) % nbuf])
        gathers[n_ch - 1].wait()
        outs[n_ch - 1] = pltpu.async_copy(
            rows_bufs[(n_ch - 1) % nbuf],
            out_hbm.at[pl.ds(base + (n_ch - 1) * ch, ch)],
            osems[(n_ch - 1) % nbuf])
        for c in range(max(0, n_ch - nbuf), n_ch):
            outs[c].wait()

    return gather_kernel(wt, idx)


def _transpose_body(q_ref, o_ref):
    o_ref[...] = jnp.transpose(q_ref[...], (0, 2, 1))


def _transpose_call(q3):
    b, p, d = q3.shape
    bs = 8
    return pl.pallas_call(
        _transpose_body,
        grid=(b // bs,),
        in_specs=[pl.BlockSpec((bs, p, d), lambda i: (i, 0, 0))],
        out_specs=pl.BlockSpec((bs, d, p), lambda i: (i, 0, 0)),
        out_shape=jax.ShapeDtypeStruct((b, d, p), jnp.float32),
    )(q3)


def kernel(x, weight):
    b, d, h, w = x.shape
    p = h * w
    x3 = x.reshape(b, d, p)
    wt = _wt_call(weight)
    idx3 = _argmin_call(x3, wt)
    quant = _sc_gather(wt, idx3.reshape(b * p))
    res3 = _transpose_call(quant.reshape(b, p, d))
    return res3.reshape(b, d, h, w), idx3.reshape(b, h, w)


# speedup vs baseline: 1.2898x; 1.0006x over previous
import functools

import jax
import jax.numpy as jnp
from jax import lax
from jax.experimental import pallas as pl
from jax.experimental.pallas import tpu as pltpu
from jax.experimental.pallas import tpu_sc as plsc


def _wt_body(w_ref, wt_ref):
    wt_ref[...] = w_ref[...].T


def _wt_call(weight):
    d, k = weight.shape
    return pl.pallas_call(
        _wt_body,
        out_shape=jax.ShapeDtypeStruct((k, d), jnp.float32),
    )(weight)


def _argmin_body(k_codes, bb, x_ref, wt_ref, idx_ref):
    wt = wt_ref[...]
    e2 = jnp.sum(wt * wt, axis=1)
    for j in range(bb):
        xb = x_ref[j]
        st = lax.dot_general(wt, xb, (((1,), (0,)), ((), ())),
                             preferred_element_type=jnp.float32)
        x2 = jnp.sum(xb * xb, axis=0)
        dist = (x2[None, :] - 2.0 * st) + e2[:, None]
        m = jnp.min(dist, axis=0, keepdims=True)
        kiota = lax.broadcasted_iota(jnp.int32, dist.shape, 0)
        am = jnp.min(jnp.where(dist == m, kiota, k_codes), axis=0)
        idx_ref[j, 0, :] = am


def _argmin_call(x3, wt):
    b, d, p = x3.shape
    k = wt.shape[0]
    bb = 2
    return pl.pallas_call(
        functools.partial(_argmin_body, k, bb),
        grid=(b // bb,),
        in_specs=[
            pl.BlockSpec((bb, d, p), lambda i: (i, 0, 0)),
            pl.BlockSpec((k, d), lambda i: (0, 0)),
        ],
        out_specs=pl.BlockSpec((bb, 1, p), lambda i: (i, 0, 0)),
        out_shape=jax.ShapeDtypeStruct((b, 1, p), jnp.int32),
    )(x3, wt)


def _sc_gather(wt, idx):
    nc, ns = 2, 16
    nw = nc * ns
    n, d = idx.shape[0], wt.shape[1]
    b_per_w = n // nw
    ch = min(64, b_per_w)
    n_ch = b_per_w // ch
    mesh = plsc.VectorSubcoreMesh(core_axis_name="c", subcore_axis_name="s",
                                  num_cores=nc, num_subcores=ns)

    nbuf = min(4, n_ch)

    @functools.partial(
        pl.kernel, mesh=mesh,
        out_type=jax.ShapeDtypeStruct((n, d), jnp.float32),
        scratch_types=(
            [pltpu.VMEM((b_per_w,), jnp.int32)]
            + [pltpu.VMEM((ch, d), jnp.float32)] * nbuf
            + [pltpu.SemaphoreType.DMA] * (2 * nbuf)
        ),
    )
    def gather_kernel(table_hbm, idx_hbm, out_hbm, idx_v, *bufs_sems):
        rows_bufs = bufs_sems[:nbuf]
        gsems = bufs_sems[nbuf:2 * nbuf]
        osems = bufs_sems[2 * nbuf:]
        wid = lax.axis_index("s") * nc + lax.axis_index("c")
        base = wid * b_per_w
        pltpu.sync_copy(idx_hbm.at[pl.ds(base, b_per_w)], idx_v)
        gathers = [None] * n_ch
        outs = [None] * n_ch
        for c in range(n_ch):
            if c >= nbuf:
                outs[c - nbuf].wait()
            gathers[c] = pltpu.async_copy(
                table_hbm.at[idx_v.at[pl.ds(c * ch, ch)]],
                rows_bufs[c % nbuf], gsems[c % nbuf])
            if c >= 1:
                gathers[c - 1].wait()
                outs[c - 1] = pltpu.async_copy(
                    rows_bufs[(c - 1) % nbuf],
                    out_hbm.at[pl.ds(base + (c - 1) * ch, ch)],
                    osems[(c - 1) % nbuf])
        gathers[n_ch - 1].wait()
        outs[n_ch - 1] = pltpu.async_copy(
            rows_bufs[(n_ch - 1) % nbuf],
            out_hbm.at[pl.ds(base + (n_ch - 1) * ch, ch)],
            osems[(n_ch - 1) % nbuf])
        for c in range(max(0, n_ch - nbuf), n_ch):
            outs[c].wait()

    return gather_kernel(wt, idx)


def _transpose_body(q_ref, o_ref):
    o_ref[...] = jnp.transpose(q_ref[...], (0, 2, 1))


def _transpose_call(q3):
    b, p, d = q3.shape
    bs = 8
    return pl.pallas_call(
        _transpose_body,
        grid=(b // bs,),
        in_specs=[pl.BlockSpec((bs, p, d), lambda i: (i, 0, 0))],
        out_specs=pl.BlockSpec((bs, d, p), lambda i: (i, 0, 0)),
        out_shape=jax.ShapeDtypeStruct((b, d, p), jnp.float32),
    )(q3)


def kernel(x, weight):
    b, d, h, w = x.shape
    p = h * w
    x3 = x.reshape(b, d, p)
    wt = _wt_call(weight)
    idx3 = _argmin_call(x3, wt)
    quant = _sc_gather(wt, idx3.reshape(b * p))
    res3 = _transpose_call(quant.reshape(b, p, d))
    return res3.reshape(b, d, h, w), idx3.reshape(b, h, w)
